# Initial kernel scaffold; baseline (speedup 1.0000x reference)
import jax
import jax.numpy as jnp
from jax.experimental import pallas as pl


def kernel(x, c0, c1, c2, Wg0, bg0, Wg1, bg1, edge_index):
    raise NotImplementedError("write your pallas kernel here")


# speedup vs baseline: 7.8030x; 7.8030x over previous
import functools

import jax
import jax.numpy as jnp
from jax import lax
from jax.experimental import pallas as pl
from jax.experimental.pallas import tpu as pltpu
from jax.experimental.pallas import tpu_sc as plsc

MIN_NORM = 1e-5
EPS = 1e-7

NC = 2
NS = 16
CHUNK = 128
BLK = 256


def _arcosh(z):
    return jnp.log(z + jnp.sqrt(jnp.maximum(z * z - 1.0, 1e-15)))


def _sinh(t):
    et = jnp.exp(t)
    return 0.5 * (et - 1.0 / et)


@functools.lru_cache(maxsize=None)
def _sc_deg(np_pad, nch):
    mesh = plsc.VectorSubcoreMesh(core_axis_name="c", subcore_axis_name="s")
    rpt = np_pad // NS

    @functools.partial(
        pl.kernel,
        out_type=jax.ShapeDtypeStruct((NC, np_pad, 16), jnp.float32),
        mesh=mesh,
        scratch_types=[
            pltpu.VMEM((rpt, 16), jnp.float32),
            pltpu.VMEM((CHUNK,), jnp.int32),
            pltpu.VMEM((CHUNK, 16), jnp.float32),
            pltpu.VMEM_SHARED((np_pad, 16), jnp.float32),
            pltpu.SemaphoreType.DMA,
        ],
        compiler_params=pltpu.CompilerParams(use_tc_tiling_on_sc=False),
    )
    def deg_kernel(row_hbm, out_hbm, ones_v, idx_v, ones128_v, acc, sem):
        cid = lax.axis_index("c")
        sid = lax.axis_index("s")
        tid = cid * NS + sid

        def fill(i, carry):
            ones_v[i, :] = jnp.full((16,), 1.0, jnp.float32)
            return carry

        lax.fori_loop(0, rpt, fill, 0)

        def fill128(i, carry):
            ones128_v[i, :] = jnp.full((16,), 1.0, jnp.float32)
            return carry

        lax.fori_loop(0, CHUNK, fill128, 0)
        pltpu.sync_copy(ones_v, acc.at[pl.ds(sid * rpt, rpt)])
        plsc.subcore_barrier()

        def chunk(k, carry):
            pltpu.sync_copy(row_hbm.at[tid, k], idx_v)
            pltpu.async_copy(ones128_v, acc.at[idx_v], sem, add=True).wait()
            return carry

        lax.fori_loop(0, nch, chunk, 0)
        plsc.subcore_barrier()
        pltpu.sync_copy(acc.at[pl.ds(sid * rpt, rpt)],
                        out_hbm.at[cid, pl.ds(sid * rpt, rpt)])

    return deg_kernel


@functools.lru_cache(maxsize=None)
def _sc_agg(np_pad, nch):
    mesh = plsc.VectorSubcoreMesh(core_axis_name="c", subcore_axis_name="s")
    rpt = np_pad // NS
    DH = 64

    @functools.partial(
        pl.kernel,
        out_type=jax.ShapeDtypeStruct((NC, np_pad, DH), jnp.float32),
        mesh=mesh,
        scratch_types=[
            pltpu.VMEM((np_pad,), jnp.float32),
            pltpu.VMEM((NC * np_pad,), jnp.float32),
            pltpu.VMEM((CHUNK,), jnp.int32),
            pltpu.VMEM((CHUNK,), jnp.int32),
            pltpu.VMEM((CHUNK,), jnp.float32),
            pltpu.VMEM((CHUNK, DH), jnp.float32),
            pltpu.VMEM_SHARED((np_pad, DH), jnp.float32),
            pltpu.SemaphoreType.DMA,
        ],
        compiler_params=pltpu.CompilerParams(needs_layout_passes=False,
                                             use_tc_tiling_on_sc=False),
    )
    def agg_kernel(xts_hbm, a_hbm, b_hbm, row_hbm, col_hbm, zero_hbm, out_hbm,
                   a_v, b_v, idxr_v, idxc_v, w_v, rows_v, acc, sem):
        cid = lax.axis_index("c")
        sid = lax.axis_index("s")
        pltpu.sync_copy(a_hbm, a_v)
        pltpu.sync_copy(b_hbm, b_v)
        pltpu.sync_copy(zero_hbm.at[pl.ds(sid * rpt, rpt)],
                        acc.at[pl.ds(sid * rpt, rpt)])
        plsc.subcore_barrier()

        def chunk(k, carry):
            pltpu.sync_copy(row_hbm.at[cid, sid, k], idxr_v)
            pltpu.sync_copy(col_hbm.at[sid, k], idxc_v)
            pltpu.async_copy(xts_hbm.at[idxr_v], rows_v, sem).wait()
            for j in range(CHUNK // 16):
                r = idxr_v[pl.ds(16 * j, 16)]
                c = idxc_v[pl.ds(16 * j, 16)]
                z = plsc.load_gather(a_v, [c]) + plsc.load_gather(b_v, [r])
                e2 = jnp.exp(z + z)
                w_v[pl.ds(16 * j, 16)] = 1.0 - 2.0 / (e2 + 1.0)

            def scale(e, carry2):
                wb = plsc.load_gather(w_v, [jnp.full((16,), 0, jnp.int32) + e])
                for f in range(DH // 16):
                    rows_v[e, pl.ds(16 * f, 16)] = rows_v[e, pl.ds(16 * f, 16)] * wb
                return carry2

            lax.fori_loop(0, CHUNK, scale, 0)
            pltpu.async_copy(rows_v, acc.at[idxc_v], sem, add=True).wait()
            return carry

        lax.fori_loop(0, nch, chunk, 0)
        plsc.subcore_barrier()
        pltpu.sync_copy(acc.at[pl.ds(sid * rpt, rpt)],
                        out_hbm.at[cid, pl.ds(sid * rpt, rpt)])

    return agg_kernel


def _tc_pre_body(x_ref, p_ref, wa_ref, wb_ref, c0_ref, bg_ref,
                 xts_ref, a_ref, b_ref, dis_ref):
    x = x_ref[...]
    cv = jax.nn.softplus(c0_ref[0, 0])
    K = 1.0 / cv
    s = jnp.sqrt(K)
    xn = jnp.maximum(jnp.sqrt(jnp.sum(x * x, 1, keepdims=True)), MIN_NORM)
    rest0 = s * _sinh(xn / s) * x / xn
    ysq = jnp.sum(rest0 * rest0, 1, keepdims=True)
    yn = jnp.maximum(jnp.sqrt(ysq), MIN_NORM)
    col0 = jnp.sqrt(jnp.maximum(K + ysq, EPS))
    xt = s * _arcosh(jnp.maximum(col0 / s, 1.0 + EPS)) * rest0 / yn
    a = jnp.sum(xt * wa_ref[...], 1, keepdims=True) + bg_ref[0, 0]
    b = jnp.sum(xt * wb_ref[...], 1, keepdims=True)
    deg = p_ref[0, :, 0:1] + p_ref[1, :, 0:1] - 1.0
    dis = 1.0 / jnp.sqrt(deg)
    xts_ref[...] = dis * xt
    a_ref[...] = a
    b_ref[...] = b
    dis_ref[...] = dis


def _post_agg(p_ref, a_ref, b_ref, dis_ref, xts_ref, Kin, sin_, Kout, sout):
    dis = dis_ref[...]
    agg = jnp.concatenate([p_ref[0], p_ref[1]], axis=1)
    m = dis * (agg + jnp.tanh(a_ref[...] + b_ref[...]) * xts_ref[...])
    mn = jnp.maximum(jnp.sqrt(jnp.sum(m * m, 1, keepdims=True)), MIN_NORM)
    rest1 = sin_ * _sinh(mn / sin_) * m / mn
    r1sq = jnp.sum(rest1 * rest1, 1, keepdims=True)
    c0a = jnp.sqrt(jnp.maximum(Kin + r1sq, EPS))
    yn1 = jnp.maximum(jnp.sqrt(r1sq), MIN_NORM)
    v = sin_ * _arcosh(jnp.maximum(c0a / sin_, 1.0 + EPS)) * rest1 / yn1
    v = jnp.maximum(v, 0.0)
    vn = jnp.maximum(jnp.sqrt(jnp.sum(v * v, 1, keepdims=True)), MIN_NORM)
    rest2 = sout * _sinh(vn / sout) * v / vn
    return dis, rest2


def _tc_mid_body(p_ref, a_ref, b_ref, dis_ref, xts_ref, wa_ref, wb_ref,
                 c0_ref, c1_ref, bg_ref, xts2_ref, a2_ref, b2_ref):
    K0 = 1.0 / jax.nn.softplus(c0_ref[0, 0])
    s0 = jnp.sqrt(K0)
    K1 = 1.0 / jax.nn.softplus(c1_ref[0, 0])
    s1 = jnp.sqrt(K1)
    dis, rest2 = _post_agg(p_ref, a_ref, b_ref, dis_ref, xts_ref, K0, s0, K1, s1)
    ysq = jnp.sum(rest2 * rest2, 1, keepdims=True)
    yn = jnp.maximum(jnp.sqrt(ysq), MIN_NORM)
    col0 = jnp.sqrt(jnp.maximum(K1 + ysq, EPS))
    xt2 = s1 * _arcosh(jnp.maximum(col0 / s1, 1.0 + EPS)) * rest2 / yn
    a2_ref[...] = jnp.sum(xt2 * wa_ref[...], 1, keepdims=True) + bg_ref[0, 0]
    b2_ref[...] = jnp.sum(xt2 * wb_ref[...], 1, keepdims=True)
    xts2_ref[...] = dis * xt2


def _tc_fin_body(p_ref, a_ref, b_ref, dis_ref, xts_ref, c1_ref, c2_ref,
                 out_ref):
    K1 = 1.0 / jax.nn.softplus(c1_ref[0, 0])
    s1 = jnp.sqrt(K1)
    K2 = 1.0 / jax.nn.softplus(c2_ref[0, 0])
    s2 = jnp.sqrt(K2)
    _, rest2 = _post_agg(p_ref, a_ref, b_ref, dis_ref, xts_ref, K1, s1, K2, s2)
    ysq = jnp.sum(rest2 * rest2, 1, keepdims=True)
    yn = jnp.maximum(jnp.sqrt(ysq), MIN_NORM)
    col0 = jnp.sqrt(jnp.maximum(K2 + ysq, EPS))
    out_ref[...] = s2 * _arcosh(jnp.maximum(col0 / s2, 1.0 + EPS)) * rest2 / yn


def _row_spec(w):
    return pl.BlockSpec((BLK, w), lambda i: (i, 0))


def _full_spec(shape):
    nd = len(shape)
    return pl.BlockSpec(shape, lambda i, _nd=nd: (0,) * _nd)


def _part_spec(w):
    return pl.BlockSpec((NC, BLK, w), lambda i: (0, i, 0))


def kernel(x, c0, c1, c2, Wg0, bg0, Wg1, bg1, edge_index):
    N, D = x.shape
    E = edge_index.shape[1]
    np_pad = ((N + BLK - 1) // BLK) * BLK
    per = NC * NS * CHUNK
    ep = ((E + per - 1) // per) * per
    nch_deg = ep // per
    nch_agg = ep // (NS * CHUNK)
    grid = (np_pad // BLK,)

    f32 = jnp.float32
    xp = jnp.pad(x.astype(f32), ((0, np_pad - N), (0, 0)))
    pad_idx = jnp.full((ep - E,), np_pad - 1, jnp.int32)
    row_p = jnp.concatenate([edge_index[0], pad_idx])
    col_p = jnp.concatenate([edge_index[1], pad_idx])
    row_deg = row_p.reshape(NC * NS, nch_deg, CHUNK)
    row_base = row_p.reshape(NS, nch_agg, CHUNK)
    row_adj = jnp.stack([row_base, row_base + np_pad])
    col_agg = col_p.reshape(NS, nch_agg, CHUNK)
    zeros_h = jnp.zeros((np_pad, 64), f32)

    wa0 = Wg0[1:129, 0].reshape(1, 128)
    wb0 = Wg0[130:258, 0].reshape(1, 128)
    wa1 = Wg1[1:129, 0].reshape(1, 128)
    wb1 = Wg1[130:258, 0].reshape(1, 128)
    c0r, c1r, c2r = c0.reshape(1, 1), c1.reshape(1, 1), c2.reshape(1, 1)
    bg0r, bg1r = bg0.reshape(1, 1), bg1.reshape(1, 1)

    degp = _sc_deg(np_pad, nch_deg)(row_deg)

    scl = pl.BlockSpec((1, 1), lambda i: (0, 0))
    xts1, a1, b1, dis = pl.pallas_call(
        _tc_pre_body,
        grid=grid,
        in_specs=[_row_spec(128), _part_spec(16), _full_spec((1, 128)),
                  _full_spec((1, 128)), scl, scl],
        out_specs=[_row_spec(128), _row_spec(1), _row_spec(1), _row_spec(1)],
        out_shape=[jax.ShapeDtypeStruct((np_pad, 128), f32),
                   jax.ShapeDtypeStruct((np_pad, 1), f32),
                   jax.ShapeDtypeStruct((np_pad, 1), f32),
                   jax.ShapeDtypeStruct((np_pad, 1), f32)],
    )(xp, degp, wa0, wb0, c0r, bg0r)

    agg = _sc_agg(np_pad, nch_agg)

    def run_agg(xts, a, b):
        xts_s = jnp.concatenate([xts[:, :64], xts[:, 64:]], axis=0)
        bb = b.reshape(np_pad)
        b_big = jnp.concatenate([bb, bb])
        return agg(xts_s, a.reshape(np_pad), b_big, row_adj, col_agg, zeros_h)

    p1 = run_agg(xts1, a1, b1)

    xts2, a2, b2 = pl.pallas_call(
        _tc_mid_body,
        grid=grid,
        in_specs=[_part_spec(64), _row_spec(1), _row_spec(1), _row_spec(1),
                  _row_spec(128), _full_spec((1, 128)), _full_spec((1, 128)),
                  scl, scl, scl],
        out_specs=[_row_spec(128), _row_spec(1), _row_spec(1)],
        out_shape=[jax.ShapeDtypeStruct((np_pad, 128), f32),
                   jax.ShapeDtypeStruct((np_pad, 1), f32),
                   jax.ShapeDtypeStruct((np_pad, 1), f32)],
    )(p1, a1, b1, dis, xts1, wa1, wb1, c0r, c1r, bg1r)

    p2 = run_agg(xts2, a2, b2)

    rest = pl.pallas_call(
        _tc_fin_body,
        grid=grid,
        in_specs=[_part_spec(64), _row_spec(1), _row_spec(1), _row_spec(1),
                  _row_spec(128), scl, scl],
        out_specs=_row_spec(128),
        out_shape=jax.ShapeDtypeStruct((np_pad, 128), f32),
    )(p2, a2, b2, dis, xts2, c1r, c2r)

    return jnp.concatenate([jnp.zeros((N, 1), f32), rest[:N]], axis=1)


# speedup vs baseline: 11.9110x; 1.5265x over previous
import functools

import jax
import jax.numpy as jnp
from jax import lax
from jax.experimental import pallas as pl
from jax.experimental.pallas import tpu as pltpu
from jax.experimental.pallas import tpu_sc as plsc

MIN_NORM = 1e-5
EPS = 1e-7

NC = 2
NS = 16
CHUNK = 128
BLK = 256


def _arcosh(z):
    return jnp.log(z + jnp.sqrt(jnp.maximum(z * z - 1.0, 1e-15)))


def _sinh(t):
    et = jnp.exp(t)
    return 0.5 * (et - 1.0 / et)


@functools.lru_cache(maxsize=None)
def _sc_deg(np_pad, nch):
    mesh = plsc.VectorSubcoreMesh(core_axis_name="c", subcore_axis_name="s")
    rpt = np_pad // NS

    @functools.partial(
        pl.kernel,
        out_type=jax.ShapeDtypeStruct((NC, np_pad, 16), jnp.float32),
        mesh=mesh,
        scratch_types=[
            pltpu.VMEM((rpt, 16), jnp.float32),
            pltpu.VMEM((CHUNK,), jnp.int32),
            pltpu.VMEM((CHUNK, 16), jnp.float32),
            pltpu.VMEM_SHARED((np_pad, 16), jnp.float32),
            pltpu.SemaphoreType.DMA,
        ],
        compiler_params=pltpu.CompilerParams(use_tc_tiling_on_sc=False),
    )
    def deg_kernel(row_hbm, out_hbm, ones_v, idx_v, ones128_v, acc, sem):
        cid = lax.axis_index("c")
        sid = lax.axis_index("s")
        tid = cid * NS + sid

        def fill(i, carry):
            ones_v[i, :] = jnp.full((16,), 1.0, jnp.float32)
            return carry

        lax.fori_loop(0, rpt, fill, 0)

        def fill128(i, carry):
            ones128_v[i, :] = jnp.full((16,), 1.0, jnp.float32)
            return carry

        lax.fori_loop(0, CHUNK, fill128, 0)
        pltpu.sync_copy(ones_v, acc.at[pl.ds(sid * rpt, rpt)])
        plsc.subcore_barrier()

        def chunk(k, carry):
            pltpu.sync_copy(row_hbm.at[tid, k], idx_v)
            pltpu.async_copy(ones128_v, acc.at[idx_v], sem, add=True).wait()
            return carry

        lax.fori_loop(0, nch, chunk, 0)
        plsc.subcore_barrier()
        pltpu.sync_copy(acc.at[pl.ds(sid * rpt, rpt)],
                        out_hbm.at[cid, pl.ds(sid * rpt, rpt)])

    return deg_kernel


NBUF = 4


@functools.lru_cache(maxsize=None)
def _sc_agg(np_pad, nch):
    mesh = plsc.VectorSubcoreMesh(core_axis_name="c", subcore_axis_name="s")
    rpt = np_pad // NS
    DH = 64
    nr = nch // NBUF

    @functools.partial(
        pl.kernel,
        out_type=jax.ShapeDtypeStruct((NC, np_pad, DH), jnp.float32),
        mesh=mesh,
        scratch_types=(
            [pltpu.VMEM((np_pad,), jnp.float32),
             pltpu.VMEM((NC * np_pad,), jnp.float32)]
            + [pltpu.VMEM((CHUNK,), jnp.int32) for _ in range(2 * NBUF)]
            + [pltpu.VMEM((CHUNK,), jnp.float32) for _ in range(NBUF)]
            + [pltpu.VMEM((CHUNK, DH), jnp.float32) for _ in range(NBUF)]
            + [pltpu.VMEM_SHARED((np_pad, DH), jnp.float32)]
            + [pltpu.SemaphoreType.DMA for _ in range(3 * NBUF)]
        ),
        compiler_params=pltpu.CompilerParams(needs_layout_passes=False,
                                             use_tc_tiling_on_sc=False),
    )
    def agg_kernel(xts_hbm, a_hbm, b_hbm, row_hbm, col_hbm, zero_hbm, out_hbm,
                   a_v, b_v, *rest):
        irs = rest[0:NBUF]
        ics = rest[NBUF:2 * NBUF]
        ws = rest[2 * NBUF:3 * NBUF]
        rows = rest[3 * NBUF:4 * NBUF]
        acc = rest[4 * NBUF]
        isems = rest[4 * NBUF + 1:4 * NBUF + 1 + NBUF]
        gsems = rest[4 * NBUF + 1 + NBUF:4 * NBUF + 1 + 2 * NBUF]
        ssems = rest[4 * NBUF + 1 + 2 * NBUF:4 * NBUF + 1 + 3 * NBUF]

        cid = lax.axis_index("c")
        sid = lax.axis_index("s")
        pltpu.sync_copy(a_hbm, a_v)
        pltpu.sync_copy(b_hbm, b_v)
        pltpu.sync_copy(zero_hbm.at[pl.ds(sid * rpt, rpt)],
                        acc.at[pl.ds(sid * rpt, rpt)])
        plsc.subcore_barrier()

        for b in range(NBUF):
            pltpu.async_copy(row_hbm.at[cid, sid, b], irs[b], isems[b])
            pltpu.async_copy(col_hbm.at[sid, b], ics[b], isems[b])

        def round_body(r, carry):
            for b in range(NBUF):
                pltpu.make_async_copy(row_hbm.at[cid, sid, 0], irs[b],
                                      isems[b]).wait()
                pltpu.make_async_copy(col_hbm.at[sid, 0], ics[b],
                                      isems[b]).wait()
                pltpu.async_copy(xts_hbm.at[irs[b]], rows[b], gsems[b])
            for b in range(NBUF):
                for j in range(CHUNK // 16):
                    rr = irs[b][pl.ds(16 * j, 16)]
                    cc = ics[b][pl.ds(16 * j, 16)]
                    z = plsc.load_gather(a_v, [cc]) + plsc.load_gather(b_v, [rr])
                    e2 = jnp.exp(z + z)
                    ws[b][pl.ds(16 * j, 16)] = 1.0 - 2.0 / (e2 + 1.0)
            for b in range(NBUF):
                pltpu.make_async_copy(xts_hbm.at[irs[b]], rows[b],
                                      gsems[b]).wait()
                _rv = rows[b]
                _wv = ws[b]

                def scale(e, _rv=_rv, _wv=_wv):
                    wb = plsc.load_gather(_wv,
                                          [jnp.full((16,), 0, jnp.int32) + e])
                    for f in range(DH // 16):
                        _rv[e, pl.ds(16 * f, 16)] = _rv[e, pl.ds(16 * f, 16)] * wb

                plsc.parallel_loop(0, CHUNK)(scale)
                pltpu.async_copy(rows[b], acc.at[ics[b]], ssems[b], add=True)
            for b in range(NBUF):
                pltpu.make_async_copy(rows[b], acc.at[ics[b]], ssems[b]).wait()

                @pl.when(r < nr - 1)
                def _prefetch(b=b, r=r):
                    pltpu.async_copy(row_hbm.at[cid, sid, (r + 1) * NBUF + b],
                                     irs[b], isems[b])
                    pltpu.async_copy(col_hbm.at[sid, (r + 1) * NBUF + b],
                                     ics[b], isems[b])
            return carry

        lax.fori_loop(0, nr, round_body, 0)
        plsc.subcore_barrier()
        pltpu.sync_copy(acc.at[pl.ds(sid * rpt, rpt)],
                        out_hbm.at[cid, pl.ds(sid * rpt, rpt)])

    return agg_kernel


def _tc_pre_body(x_ref, p_ref, wa_ref, wb_ref, c0_ref, bg_ref,
                 xts_ref, a_ref, b_ref, dis_ref):
    x = x_ref[...]
    cv = jax.nn.softplus(c0_ref[0, 0])
    K = 1.0 / cv
    s = jnp.sqrt(K)
    xn = jnp.maximum(jnp.sqrt(jnp.sum(x * x, 1, keepdims=True)), MIN_NORM)
    rest0 = s * _sinh(xn / s) * x / xn
    ysq = jnp.sum(rest0 * rest0, 1, keepdims=True)
    yn = jnp.maximum(jnp.sqrt(ysq), MIN_NORM)
    col0 = jnp.sqrt(jnp.maximum(K + ysq, EPS))
    xt = s * _arcosh(jnp.maximum(col0 / s, 1.0 + EPS)) * rest0 / yn
    a = jnp.sum(xt * wa_ref[...], 1, keepdims=True) + bg_ref[0, 0]
    b = jnp.sum(xt * wb_ref[...], 1, keepdims=True)
    deg = p_ref[0, :, 0:1] + p_ref[1, :, 0:1] - 1.0
    dis = 1.0 / jnp.sqrt(deg)
    xts_ref[...] = dis * xt
    a_ref[...] = a
    b_ref[...] = b
    dis_ref[...] = dis


def _post_agg(p_ref, a_ref, b_ref, dis_ref, xts_ref, Kin, sin_, Kout, sout):
    dis = dis_ref[...]
    agg = jnp.concatenate([p_ref[0], p_ref[1]], axis=1)
    m = dis * (agg + jnp.tanh(a_ref[...] + b_ref[...]) * xts_ref[...])
    mn = jnp.maximum(jnp.sqrt(jnp.sum(m * m, 1, keepdims=True)), MIN_NORM)
    rest1 = sin_ * _sinh(mn / sin_) * m / mn
    r1sq = jnp.sum(rest1 * rest1, 1, keepdims=True)
    c0a = jnp.sqrt(jnp.maximum(Kin + r1sq, EPS))
    yn1 = jnp.maximum(jnp.sqrt(r1sq), MIN_NORM)
    v = sin_ * _arcosh(jnp.maximum(c0a / sin_, 1.0 + EPS)) * rest1 / yn1
    v = jnp.maximum(v, 0.0)
    vn = jnp.maximum(jnp.sqrt(jnp.sum(v * v, 1, keepdims=True)), MIN_NORM)
    rest2 = sout * _sinh(vn / sout) * v / vn
    return dis, rest2


def _tc_mid_body(p_ref, a_ref, b_ref, dis_ref, xts_ref, wa_ref, wb_ref,
                 c0_ref, c1_ref, bg_ref, xts2_ref, a2_ref, b2_ref):
    K0 = 1.0 / jax.nn.softplus(c0_ref[0, 0])
    s0 = jnp.sqrt(K0)
    K1 = 1.0 / jax.nn.softplus(c1_ref[0, 0])
    s1 = jnp.sqrt(K1)
    dis, rest2 = _post_agg(p_ref, a_ref, b_ref, dis_ref, xts_ref, K0, s0, K1, s1)
    ysq = jnp.sum(rest2 * rest2, 1, keepdims=True)
    yn = jnp.maximum(jnp.sqrt(ysq), MIN_NORM)
    col0 = jnp.sqrt(jnp.maximum(K1 + ysq, EPS))
    xt2 = s1 * _arcosh(jnp.maximum(col0 / s1, 1.0 + EPS)) * rest2 / yn
    a2_ref[...] = jnp.sum(xt2 * wa_ref[...], 1, keepdims=True) + bg_ref[0, 0]
    b2_ref[...] = jnp.sum(xt2 * wb_ref[...], 1, keepdims=True)
    xts2_ref[...] = dis * xt2


def _tc_fin_body(p_ref, a_ref, b_ref, dis_ref, xts_ref, c1_ref, c2_ref,
                 out_ref):
    K1 = 1.0 / jax.nn.softplus(c1_ref[0, 0])
    s1 = jnp.sqrt(K1)
    K2 = 1.0 / jax.nn.softplus(c2_ref[0, 0])
    s2 = jnp.sqrt(K2)
    _, rest2 = _post_agg(p_ref, a_ref, b_ref, dis_ref, xts_ref, K1, s1, K2, s2)
    ysq = jnp.sum(rest2 * rest2, 1, keepdims=True)
    yn = jnp.maximum(jnp.sqrt(ysq), MIN_NORM)
    col0 = jnp.sqrt(jnp.maximum(K2 + ysq, EPS))
    out_ref[...] = s2 * _arcosh(jnp.maximum(col0 / s2, 1.0 + EPS)) * rest2 / yn


def _row_spec(w):
    return pl.BlockSpec((BLK, w), lambda i: (i, 0))


def _full_spec(shape):
    nd = len(shape)
    return pl.BlockSpec(shape, lambda i, _nd=nd: (0,) * _nd)


def _part_spec(w):
    return pl.BlockSpec((NC, BLK, w), lambda i: (0, i, 0))


def kernel(x, c0, c1, c2, Wg0, bg0, Wg1, bg1, edge_index):
    N, D = x.shape
    E = edge_index.shape[1]
    np_pad = ((N + BLK - 1) // BLK) * BLK
    per = NS * CHUNK * NBUF
    assert per % (NC * NS * CHUNK) == 0
    ep = ((E + per - 1) // per) * per
    nch_deg = ep // (NC * NS * CHUNK)
    nch_agg = ep // (NS * CHUNK)
    grid = (np_pad // BLK,)

    f32 = jnp.float32
    xp = jnp.pad(x.astype(f32), ((0, np_pad - N), (0, 0)))
    pad_idx = jnp.full((ep - E,), np_pad - 1, jnp.int32)
    row_p = jnp.concatenate([edge_index[0], pad_idx])
    col_p = jnp.concatenate([edge_index[1], pad_idx])
    row_deg = row_p.reshape(NC * NS, nch_deg, CHUNK)
    row_base = row_p.reshape(NS, nch_agg, CHUNK)
    row_adj = jnp.stack([row_base, row_base + np_pad])
    col_agg = col_p.reshape(NS, nch_agg, CHUNK)
    zeros_h = jnp.zeros((np_pad, 64), f32)

    wa0 = Wg0[1:129, 0].reshape(1, 128)
    wb0 = Wg0[130:258, 0].reshape(1, 128)
    wa1 = Wg1[1:129, 0].reshape(1, 128)
    wb1 = Wg1[130:258, 0].reshape(1, 128)
    c0r, c1r, c2r = c0.reshape(1, 1), c1.reshape(1, 1), c2.reshape(1, 1)
    bg0r, bg1r = bg0.reshape(1, 1), bg1.reshape(1, 1)

    degp = _sc_deg(np_pad, nch_deg)(row_deg)

    scl = pl.BlockSpec((1, 1), lambda i: (0, 0))
    xts1, a1, b1, dis = pl.pallas_call(
        _tc_pre_body,
        grid=grid,
        in_specs=[_row_spec(128), _part_spec(16), _full_spec((1, 128)),
                  _full_spec((1, 128)), scl, scl],
        out_specs=[_row_spec(128), _row_spec(1), _row_spec(1), _row_spec(1)],
        out_shape=[jax.ShapeDtypeStruct((np_pad, 128), f32),
                   jax.ShapeDtypeStruct((np_pad, 1), f32),
                   jax.ShapeDtypeStruct((np_pad, 1), f32),
                   jax.ShapeDtypeStruct((np_pad, 1), f32)],
    )(xp, degp, wa0, wb0, c0r, bg0r)

    agg = _sc_agg(np_pad, nch_agg)

    def run_agg(xts, a, b):
        xts_s = jnp.concatenate([xts[:, :64], xts[:, 64:]], axis=0)
        bb = b.reshape(np_pad)
        b_big = jnp.concatenate([bb, bb])
        return agg(xts_s, a.reshape(np_pad), b_big, row_adj, col_agg, zeros_h)

    p1 = run_agg(xts1, a1, b1)

    xts2, a2, b2 = pl.pallas_call(
        _tc_mid_body,
        grid=grid,
        in_specs=[_part_spec(64), _row_spec(1), _row_spec(1), _row_spec(1),
                  _row_spec(128), _full_spec((1, 128)), _full_spec((1, 128)),
                  scl, scl, scl],
        out_specs=[_row_spec(128), _row_spec(1), _row_spec(1)],
        out_shape=[jax.ShapeDtypeStruct((np_pad, 128), f32),
                   jax.ShapeDtypeStruct((np_pad, 1), f32),
                   jax.ShapeDtypeStruct((np_pad, 1), f32)],
    )(p1, a1, b1, dis, xts1, wa1, wb1, c0r, c1r, bg1r)

    p2 = run_agg(xts2, a2, b2)

    rest = pl.pallas_call(
        _tc_fin_body,
        grid=grid,
        in_specs=[_part_spec(64), _row_spec(1), _row_spec(1), _row_spec(1),
                  _row_spec(128), scl, scl],
        out_specs=_row_spec(128),
        out_shape=jax.ShapeDtypeStruct((np_pad, 128), f32),
    )(p2, a2, b2, dis, xts2, c1r, c2r)

    return jnp.concatenate([jnp.zeros((N, 1), f32), rest[:N]], axis=1)


# speedup vs baseline: 12.3467x; 1.0366x over previous
import functools

import jax
import jax.numpy as jnp
from jax import lax
from jax.experimental import pallas as pl
from jax.experimental.pallas import tpu as pltpu
from jax.experimental.pallas import tpu_sc as plsc

MIN_NORM = 1e-5
EPS = 1e-7

NC = 2
NS = 16
CHUNK = 128
BLK = 256


def _arcosh(z):
    return jnp.log(z + jnp.sqrt(jnp.maximum(z * z - 1.0, 1e-15)))


def _sinh(t):
    et = jnp.exp(t)
    return 0.5 * (et - 1.0 / et)


@functools.lru_cache(maxsize=None)
def _sc_deg(np_pad, nch):
    mesh = plsc.VectorSubcoreMesh(core_axis_name="c", subcore_axis_name="s")
    rpt = np_pad // NS

    @functools.partial(
        pl.kernel,
        out_type=jax.ShapeDtypeStruct((NC, np_pad, 16), jnp.float32),
        mesh=mesh,
        scratch_types=[
            pltpu.VMEM((rpt, 16), jnp.float32),
            pltpu.VMEM((CHUNK,), jnp.int32),
            pltpu.VMEM((CHUNK, 16), jnp.float32),
            pltpu.VMEM_SHARED((np_pad, 16), jnp.float32),
            pltpu.SemaphoreType.DMA,
        ],
        compiler_params=pltpu.CompilerParams(use_tc_tiling_on_sc=False),
    )
    def deg_kernel(row_hbm, out_hbm, ones_v, idx_v, ones128_v, acc, sem):
        cid = lax.axis_index("c")
        sid = lax.axis_index("s")
        tid = cid * NS + sid

        def fill(i, carry):
            ones_v[i, :] = jnp.full((16,), 1.0, jnp.float32)
            return carry

        lax.fori_loop(0, rpt, fill, 0)

        def fill128(i, carry):
            ones128_v[i, :] = jnp.full((16,), 1.0, jnp.float32)
            return carry

        lax.fori_loop(0, CHUNK, fill128, 0)
        pltpu.sync_copy(ones_v, acc.at[pl.ds(sid * rpt, rpt)])
        plsc.subcore_barrier()

        def chunk(k, carry):
            pltpu.sync_copy(row_hbm.at[tid, k], idx_v)
            pltpu.async_copy(ones128_v, acc.at[idx_v], sem, add=True).wait()
            return carry

        lax.fori_loop(0, nch, chunk, 0)
        plsc.subcore_barrier()
        pltpu.sync_copy(acc.at[pl.ds(sid * rpt, rpt)],
                        out_hbm.at[cid, pl.ds(sid * rpt, rpt)])

    return deg_kernel


NBUF = 4


@functools.lru_cache(maxsize=None)
def _sc_agg(np_pad, nch):
    mesh = plsc.VectorSubcoreMesh(core_axis_name="c", subcore_axis_name="s")
    rpt = np_pad // NS
    DH = 64
    nr = nch // NBUF

    @functools.partial(
        pl.kernel,
        out_type=jax.ShapeDtypeStruct((NC, np_pad, DH), jnp.float32),
        mesh=mesh,
        scratch_types=(
            [pltpu.VMEM((np_pad,), jnp.float32),
             pltpu.VMEM((NC * np_pad,), jnp.float32)]
            + [pltpu.VMEM((CHUNK,), jnp.int32) for _ in range(2 * NBUF)]
            + [pltpu.VMEM((CHUNK,), jnp.float32) for _ in range(NBUF)]
            + [pltpu.VMEM((CHUNK, DH), jnp.float32) for _ in range(NBUF)]
            + [pltpu.VMEM_SHARED((np_pad, DH), jnp.float32)]
            + [pltpu.SemaphoreType.DMA for _ in range(3 * NBUF)]
        ),
        compiler_params=pltpu.CompilerParams(needs_layout_passes=False,
                                             use_tc_tiling_on_sc=False),
    )
    def agg_kernel(xts_hbm, a_hbm, b_hbm, row_hbm, col_hbm, zero_hbm, out_hbm,
                   a_v, b_v, *rest):
        irs = rest[0:NBUF]
        ics = rest[NBUF:2 * NBUF]
        ws = rest[2 * NBUF:3 * NBUF]
        rows = rest[3 * NBUF:4 * NBUF]
        acc = rest[4 * NBUF]
        isems = rest[4 * NBUF + 1:4 * NBUF + 1 + NBUF]
        gsems = rest[4 * NBUF + 1 + NBUF:4 * NBUF + 1 + 2 * NBUF]
        ssems = rest[4 * NBUF + 1 + 2 * NBUF:4 * NBUF + 1 + 3 * NBUF]

        cid = lax.axis_index("c")
        sid = lax.axis_index("s")
        pltpu.sync_copy(a_hbm, a_v)
        pltpu.sync_copy(b_hbm, b_v)
        pltpu.sync_copy(zero_hbm.at[pl.ds(sid * rpt, rpt)],
                        acc.at[pl.ds(sid * rpt, rpt)])
        plsc.subcore_barrier()

        for b in range(NBUF):
            pltpu.async_copy(row_hbm.at[sid, b], irs[b], isems[b])
            pltpu.async_copy(col_hbm.at[sid, b], ics[b], isems[b])

        off = cid * np_pad

        def round_body(r, carry):
            for b in range(NBUF):
                pltpu.make_async_copy(row_hbm.at[sid, 0], irs[b],
                                      isems[b]).wait()
                pltpu.make_async_copy(col_hbm.at[sid, 0], ics[b],
                                      isems[b]).wait()
                for j in range(CHUNK // 16):
                    sl = pl.ds(16 * j, 16)
                    irs[b][sl] = irs[b][sl] + off
                pltpu.async_copy(xts_hbm.at[irs[b]], rows[b], gsems[b])
            for b in range(NBUF):
                for j in range(CHUNK // 16):
                    rr = irs[b][pl.ds(16 * j, 16)]
                    cc = ics[b][pl.ds(16 * j, 16)]
                    z = plsc.load_gather(a_v, [cc]) + plsc.load_gather(b_v, [rr])
                    e2 = jnp.exp(z + z)
                    ws[b][pl.ds(16 * j, 16)] = 1.0 - 2.0 / (e2 + 1.0)
            for b in range(NBUF):
                pltpu.make_async_copy(xts_hbm.at[irs[b]], rows[b],
                                      gsems[b]).wait()
                _rv = rows[b]
                _wv = ws[b]

                def scale(e, _rv=_rv, _wv=_wv):
                    wb = plsc.load_gather(_wv,
                                          [jnp.full((16,), 0, jnp.int32) + e])
                    for f in range(DH // 16):
                        _rv[e, pl.ds(16 * f, 16)] = _rv[e, pl.ds(16 * f, 16)] * wb

                plsc.parallel_loop(0, CHUNK, unroll=4)(scale)
                pltpu.async_copy(rows[b], acc.at[ics[b]], ssems[b], add=True)
            for b in range(NBUF):
                pltpu.make_async_copy(rows[b], acc.at[ics[b]], ssems[b]).wait()

                @pl.when(r < nr - 1)
                def _prefetch(b=b, r=r):
                    pltpu.async_copy(row_hbm.at[sid, (r + 1) * NBUF + b],
                                     irs[b], isems[b])
                    pltpu.async_copy(col_hbm.at[sid, (r + 1) * NBUF + b],
                                     ics[b], isems[b])
            return carry

        lax.fori_loop(0, nr, round_body, 0)
        plsc.subcore_barrier()
        pltpu.sync_copy(acc.at[pl.ds(sid * rpt, rpt)],
                        out_hbm.at[cid, pl.ds(sid * rpt, rpt)])

    return agg_kernel


def _tc_pre_body(x_ref, p_ref, wa_ref, wb_ref, c0_ref, bg_ref,
                 xts_ref, a_ref, b_ref, dis_ref):
    x = x_ref[...]
    cv = jax.nn.softplus(c0_ref[0, 0])
    K = 1.0 / cv
    s = jnp.sqrt(K)
    xn = jnp.maximum(jnp.sqrt(jnp.sum(x * x, 1, keepdims=True)), MIN_NORM)
    rest0 = s * _sinh(xn / s) * x / xn
    ysq = jnp.sum(rest0 * rest0, 1, keepdims=True)
    yn = jnp.maximum(jnp.sqrt(ysq), MIN_NORM)
    col0 = jnp.sqrt(jnp.maximum(K + ysq, EPS))
    xt = s * _arcosh(jnp.maximum(col0 / s, 1.0 + EPS)) * rest0 / yn
    a = jnp.sum(xt * wa_ref[...], 1, keepdims=True) + bg_ref[0, 0]
    b = jnp.sum(xt * wb_ref[...], 1, keepdims=True)
    deg = p_ref[0, :, 0:1] + p_ref[1, :, 0:1] - 1.0
    dis = 1.0 / jnp.sqrt(deg)
    xts = dis * xt
    xts_ref[0] = xts[:, :64]
    xts_ref[1] = xts[:, 64:]
    a_ref[...] = a
    b_ref[...] = b
    dis_ref[...] = dis


def _post_agg(p_ref, a_ref, b_ref, dis_ref, xts_ref, Kin, sin_, Kout, sout):
    dis = dis_ref[...]
    agg = jnp.concatenate([p_ref[0], p_ref[1]], axis=1)
    xts = jnp.concatenate([xts_ref[0], xts_ref[1]], axis=1)
    m = dis * (agg + jnp.tanh(a_ref[...] + b_ref[...]) * xts)
    mn = jnp.maximum(jnp.sqrt(jnp.sum(m * m, 1, keepdims=True)), MIN_NORM)
    rest1 = sin_ * _sinh(mn / sin_) * m / mn
    r1sq = jnp.sum(rest1 * rest1, 1, keepdims=True)
    c0a = jnp.sqrt(jnp.maximum(Kin + r1sq, EPS))
    yn1 = jnp.maximum(jnp.sqrt(r1sq), MIN_NORM)
    v = sin_ * _arcosh(jnp.maximum(c0a / sin_, 1.0 + EPS)) * rest1 / yn1
    v = jnp.maximum(v, 0.0)
    vn = jnp.maximum(jnp.sqrt(jnp.sum(v * v, 1, keepdims=True)), MIN_NORM)
    rest2 = sout * _sinh(vn / sout) * v / vn
    return dis, rest2


def _tc_mid_body(p_ref, a_ref, b_ref, dis_ref, xts_ref, wa_ref, wb_ref,
                 c0_ref, c1_ref, bg_ref, xts2_ref, a2_ref, b2_ref):
    K0 = 1.0 / jax.nn.softplus(c0_ref[0, 0])
    s0 = jnp.sqrt(K0)
    K1 = 1.0 / jax.nn.softplus(c1_ref[0, 0])
    s1 = jnp.sqrt(K1)
    dis, rest2 = _post_agg(p_ref, a_ref, b_ref, dis_ref, xts_ref, K0, s0, K1, s1)
    ysq = jnp.sum(rest2 * rest2, 1, keepdims=True)
    yn = jnp.maximum(jnp.sqrt(ysq), MIN_NORM)
    col0 = jnp.sqrt(jnp.maximum(K1 + ysq, EPS))
    xt2 = s1 * _arcosh(jnp.maximum(col0 / s1, 1.0 + EPS)) * rest2 / yn
    a2_ref[...] = jnp.sum(xt2 * wa_ref[...], 1, keepdims=True) + bg_ref[0, 0]
    b2_ref[...] = jnp.sum(xt2 * wb_ref[...], 1, keepdims=True)
    xts2 = dis * xt2
    xts2_ref[0] = xts2[:, :64]
    xts2_ref[1] = xts2[:, 64:]


def _tc_fin_body(p_ref, a_ref, b_ref, dis_ref, xts_ref, c1_ref, c2_ref,
                 out_ref):
    K1 = 1.0 / jax.nn.softplus(c1_ref[0, 0])
    s1 = jnp.sqrt(K1)
    K2 = 1.0 / jax.nn.softplus(c2_ref[0, 0])
    s2 = jnp.sqrt(K2)
    _, rest2 = _post_agg(p_ref, a_ref, b_ref, dis_ref, xts_ref, K1, s1, K2, s2)
    ysq = jnp.sum(rest2 * rest2, 1, keepdims=True)
    yn = jnp.maximum(jnp.sqrt(ysq), MIN_NORM)
    col0 = jnp.sqrt(jnp.maximum(K2 + ysq, EPS))
    out_ref[...] = s2 * _arcosh(jnp.maximum(col0 / s2, 1.0 + EPS)) * rest2 / yn


def _row_spec(w):
    return pl.BlockSpec((BLK, w), lambda i: (i, 0))


def _full_spec(shape):
    nd = len(shape)
    return pl.BlockSpec(shape, lambda i, _nd=nd: (0,) * _nd)


def _part_spec(w):
    return pl.BlockSpec((NC, BLK, w), lambda i: (0, i, 0))


def kernel(x, c0, c1, c2, Wg0, bg0, Wg1, bg1, edge_index):
    N, D = x.shape
    E = edge_index.shape[1]
    np_pad = ((N + BLK - 1) // BLK) * BLK
    per = NS * CHUNK * NBUF
    assert per % (NC * NS * CHUNK) == 0
    ep = ((E + per - 1) // per) * per
    nch_deg = ep // (NC * NS * CHUNK)
    nch_agg = ep // (NS * CHUNK)
    grid = (np_pad // BLK,)

    f32 = jnp.float32
    xp = jnp.pad(x.astype(f32), ((0, np_pad - N), (0, 0)))
    pad_idx = jnp.full((ep - E,), np_pad - 1, jnp.int32)
    row_p = jnp.concatenate([edge_index[0], pad_idx])
    col_p = jnp.concatenate([edge_index[1], pad_idx])
    row_deg = row_p.reshape(NC * NS, nch_deg, CHUNK)
    row_base = row_p.reshape(NS, nch_agg, CHUNK)
    col_agg = col_p.reshape(NS, nch_agg, CHUNK)
    zeros_h = jnp.zeros((np_pad, 64), f32)

    wa0 = Wg0[1:129, 0].reshape(1, 128)
    wb0 = Wg0[130:258, 0].reshape(1, 128)
    wa1 = Wg1[1:129, 0].reshape(1, 128)
    wb1 = Wg1[130:258, 0].reshape(1, 128)
    c0r, c1r, c2r = c0.reshape(1, 1), c1.reshape(1, 1), c2.reshape(1, 1)
    bg0r, bg1r = bg0.reshape(1, 1), bg1.reshape(1, 1)

    degp = _sc_deg(np_pad, nch_deg)(row_deg)

    scl = pl.BlockSpec((1, 1), lambda i: (0, 0))
    xts1, a1, b1, dis = pl.pallas_call(
        _tc_pre_body,
        grid=grid,
        in_specs=[_row_spec(128), _part_spec(16), _full_spec((1, 128)),
                  _full_spec((1, 128)), scl, scl],
        out_specs=[_part_spec(64), _row_spec(1), _row_spec(1), _row_spec(1)],
        out_shape=[jax.ShapeDtypeStruct((NC, np_pad, 64), f32),
                   jax.ShapeDtypeStruct((np_pad, 1), f32),
                   jax.ShapeDtypeStruct((np_pad, 1), f32),
                   jax.ShapeDtypeStruct((np_pad, 1), f32)],
    )(xp, degp, wa0, wb0, c0r, bg0r)

    agg = _sc_agg(np_pad, nch_agg)

    def run_agg(xts_stacked, a, b):
        xts_s = xts_stacked.reshape(NC * np_pad, 64)
        bb = b.reshape(np_pad)
        b_big = jnp.concatenate([bb, bb])
        return agg(xts_s, a.reshape(np_pad), b_big, row_base, col_agg, zeros_h)

    p1 = run_agg(xts1, a1, b1)

    xts2, a2, b2 = pl.pallas_call(
        _tc_mid_body,
        grid=grid,
        in_specs=[_part_spec(64), _row_spec(1), _row_spec(1), _row_spec(1),
                  _part_spec(64), _full_spec((1, 128)), _full_spec((1, 128)),
                  scl, scl, scl],
        out_specs=[_part_spec(64), _row_spec(1), _row_spec(1)],
        out_shape=[jax.ShapeDtypeStruct((NC, np_pad, 64), f32),
                   jax.ShapeDtypeStruct((np_pad, 1), f32),
                   jax.ShapeDtypeStruct((np_pad, 1), f32)],
    )(p1, a1, b1, dis, xts1, wa1, wb1, c0r, c1r, bg1r)

    p2 = run_agg(xts2, a2, b2)

    rest = pl.pallas_call(
        _tc_fin_body,
        grid=grid,
        in_specs=[_part_spec(64), _row_spec(1), _row_spec(1), _row_spec(1),
                  _part_spec(64), scl, scl],
        out_specs=_row_spec(128),
        out_shape=jax.ShapeDtypeStruct((np_pad, 128), f32),
    )(p2, a2, b2, dis, xts2, c1r, c2r)

    return jnp.concatenate([jnp.zeros((N, 1), f32), rest[:N]], axis=1)
